# baseline (speedup 1.0000x reference)
import jax
import jax.numpy as jnp
from jax import lax
from jax.experimental import pallas as pl
from jax.experimental.pallas import tpu as pltpu
from jax.experimental.pallas import tpu_sc as plsc

B = 16384
DIM = 32
NI = 5
NC = 2
NS = 16
NW = NC * NS
BPW = B // NW
SB = 128
NSB = BPW // SB
CHUNK = 128
PACK = 128 // DIM
NIDS = 1000000
TROWS = NIDS // PACK
CW = 1024
NCH = NIDS // CW
CPW = 31
TAILC = NCH

_params = pltpu.CompilerParams(use_tc_tiling_on_sc=True,
                               needs_layout_passes=False)


def _relayout_body(utabT, itabT, utail, itail, urm, irm, chunk_v, out_v, sem):
    wid = lax.axis_index("s") * NC + lax.axis_index("c")
    iota16 = lax.broadcasted_iota(jnp.int32, (16,), 0)
    rowv = [iota16, iota16 + 16]

    UNROLL = 8

    def transpose_rows(nrows):
        def krow(kq, carry):
            for kk in range(UNROLL):
                k = kq * UNROLL + kk
                for s in range(8):
                    colv = jnp.zeros((16,), jnp.int32) + (k * PACK + (s >> 1))
                    v = plsc.load_gather(chunk_v, [rowv[s & 1], colv])
                    out_v[k, pl.ds(s * 16, 16)] = v
            return carry
        lax.fori_loop(0, nrows // UNROLL, krow, 0)

    for tab_hbm, tail_hbm, out_hbm in ((utabT, utail, urm),
                                       (itabT, itail, irm)):
        def chunk_step(i, carry):
            kc = wid * CPW + i

            @pl.when(kc < NCH)
            def _full():
                off = pl.multiple_of(kc * CW, CW)
                pltpu.sync_copy(tab_hbm.at[:, pl.ds(off, CW)], chunk_v)
                transpose_rows(CW // PACK)
                pltpu.sync_copy(out_v,
                                out_hbm.at[pl.ds(kc * (CW // PACK), CW // PACK)])

            @pl.when(kc == TAILC)
            def _partial():
                off = pl.multiple_of(TAILC * CW, CW)
                pltpu.sync_copy(tab_hbm.at[:, pl.ds(off, 512)],
                                chunk_v.at[:, pl.ds(0, 512)])
                transpose_rows(512 // PACK)
                pltpu.sync_copy(out_v.at[pl.ds(0, 128)],
                                out_hbm.at[pl.ds(TAILC * (CW // PACK), 128)])
                pltpu.sync_copy(tail_hbm, out_v.at[pl.ds(0, 16)])
                pltpu.sync_copy(out_v.at[pl.ds(0, 16)],
                                out_hbm.at[pl.ds(TROWS - 16, 16)])

            return carry

        lax.fori_loop(0, CPW, chunk_step, 0)


_relayout_call = pl.kernel(
    _relayout_body,
    mesh=plsc.VectorSubcoreMesh(core_axis_name="c", subcore_axis_name="s"),
    compiler_params=_params,
    out_type=[
        jax.ShapeDtypeStruct((TROWS, 128), jnp.float32),
        jax.ShapeDtypeStruct((TROWS, 128), jnp.float32),
    ],
    scratch_types=[
        pltpu.VMEM((DIM, CW), jnp.float32),
        pltpu.VMEM((CW // PACK, 128), jnp.float32),
        pltpu.SemaphoreType.DMA,
    ],
)


def _sc_body(uid_hbm, ids_hbm, utab_hbm, itab_hbm, s_hbm, d0_hbm,
             uidx_v, iidx_v, ubidx_v, ibidx_v, urows_v, irows_v,
             s_v, d0_v, sem):
    wid = lax.axis_index("s") * NC + lax.axis_index("c")
    base = wid * BPW

    pltpu.sync_copy(uid_hbm.at[pl.ds(base, BPW)], uidx_v)
    pltpu.sync_copy(ids_hbm.at[pl.ds(base * NI, BPW * NI)], iidx_v)
    for v in range(BPW // 16):
        ubidx_v[pl.ds(v * 16, 16)] = jnp.right_shift(uidx_v[pl.ds(v * 16, 16)], 2)
    def shift_items(v, carry):
        ibidx_v[pl.ds(v * 16, 16)] = jnp.right_shift(iidx_v[pl.ds(v * 16, 16)], 2)
        return carry
    lax.fori_loop(0, BPW * NI // 16, shift_items, 0)

    iota16 = lax.broadcasted_iota(jnp.int32, (16,), 0)

    for sb in range(NSB):
        copies = [pltpu.async_copy(
            utab_hbm.at[ubidx_v.at[pl.ds(sb * SB, CHUNK)]], urows_v, sem)]
        for c in range(SB * NI // CHUNK):
            copies.append(pltpu.async_copy(
                itab_hbm.at[ibidx_v.at[pl.ds(sb * SB * NI + c * CHUNK, CHUNK)]],
                irows_v.at[pl.ds(c * CHUNK, CHUNK)], sem))
        for cp in copies:
            cp.wait()

        def group(g, carry):
            lrow = g * 16 + iota16
            grow = sb * SB + g * 16 + iota16
            uids = plsc.load_gather(uidx_v, [grow])
            ucol = (uids & 3) * DIM
            icols = []
            irows = []
            for j in range(NI):
                pos = grow * NI + j
                ids_j = plsc.load_gather(iidx_v, [pos])
                icols.append((ids_j & 3) * DIM)
                irows.append(lrow * NI + j)
            acc = [jnp.zeros((16,), jnp.float32) for _ in range(NI)]
            for d in range(DIM):
                u = plsc.load_gather(urows_v, [lrow, ucol + d])
                for j in range(NI):
                    it = plsc.load_gather(irows_v, [irows[j], icols[j] + d])
                    acc[j] = acc[j] + u * it
            ssum = jnp.exp(acc[0])
            for j in range(1, NI):
                ssum = ssum + jnp.exp(acc[j])
            s_v[pl.ds(sb * SB + g * 16, 16)] = ssum
            d0_v[pl.ds(sb * SB + g * 16, 16)] = acc[0]
            return carry

        lax.fori_loop(0, SB // 16, group, 0)

    pltpu.sync_copy(s_v, s_hbm.at[pl.ds(base, BPW)])
    pltpu.sync_copy(d0_v, d0_hbm.at[pl.ds(base, BPW)])


_sc_call = pl.kernel(
    _sc_body,
    mesh=plsc.VectorSubcoreMesh(core_axis_name="c", subcore_axis_name="s"),
    compiler_params=_params,
    out_type=[
        jax.ShapeDtypeStruct((B,), jnp.float32),
        jax.ShapeDtypeStruct((B,), jnp.float32),
    ],
    scratch_types=[
        pltpu.VMEM((BPW,), jnp.int32),
        pltpu.VMEM((BPW * NI,), jnp.int32),
        pltpu.VMEM((BPW,), jnp.int32),
        pltpu.VMEM((BPW * NI,), jnp.int32),
        pltpu.VMEM((SB, 128), jnp.float32),
        pltpu.VMEM((SB * NI, 128), jnp.float32),
        pltpu.VMEM((BPW,), jnp.float32),
        pltpu.VMEM((BPW,), jnp.float32),
        pltpu.SemaphoreType.DMA,
    ],
)


def _tc_loss_body(s_ref, d0_ref, out_ref):
    out_ref[0, 0] = (jnp.sum(jnp.log(s_ref[:])) - jnp.sum(d0_ref[:])) / B


_tc_loss = pl.pallas_call(
    _tc_loss_body,
    out_shape=jax.ShapeDtypeStruct((1, 1), jnp.float32),
    out_specs=pl.BlockSpec(memory_space=pltpu.SMEM),
)


def kernel(userid, itemid, user_feature, item_feature, neg_sample,
           user_table, item_table):
    uid = userid.reshape(B).astype(jnp.int32)
    ids = jnp.concatenate(
        [itemid.astype(jnp.int32), neg_sample.astype(jnp.int32)], axis=1
    ).reshape(B * NI)
    utail = lax.slice(user_table, (NIDS - 64, 0), (NIDS, DIM)).reshape(16, 128)
    itail = lax.slice(item_table, (NIDS - 64, 0), (NIDS, DIM)).reshape(16, 128)
    urm, irm = _relayout_call(user_table.T, item_table.T, utail, itail)
    s, d0 = _sc_call(uid, ids, urm, irm)
    loss = _tc_loss(s.reshape(B // 128, 128), d0.reshape(B // 128, 128))
    return loss[0, 0]


# speedup vs baseline: 1.0001x; 1.0001x over previous
import jax
import jax.numpy as jnp
from jax import lax
from jax.experimental import pallas as pl
from jax.experimental.pallas import tpu as pltpu
from jax.experimental.pallas import tpu_sc as plsc

B = 16384
DIM = 32
NI = 5
NC = 2
NS = 16
NW = NC * NS
BPW = B // NW
SB = 128
NSB = BPW // SB
CHUNK = 128
PACK = 128 // DIM
NIDS = 1000000
TROWS = NIDS // PACK
CW = 1024
NCH = NIDS // CW
CPW = 31
TAILC = NCH

_params = pltpu.CompilerParams(use_tc_tiling_on_sc=True,
                               needs_layout_passes=False)


def _relayout_body(utabT, itabT, utail, itail, urm, irm, chunk_v, out_v, sem):
    wid = lax.axis_index("s") * NC + lax.axis_index("c")
    iota16 = lax.broadcasted_iota(jnp.int32, (16,), 0)
    rowv = [iota16, iota16 + 16]

    UNROLL = 8

    def transpose_rows(nrows):
        def krow(kq, carry):
            for kk in range(UNROLL):
                k = kq * UNROLL + kk
                for s in range(8):
                    colv = jnp.zeros((16,), jnp.int32) + (k * PACK + (s >> 1))
                    v = plsc.load_gather(chunk_v, [rowv[s & 1], colv])
                    out_v[k, pl.ds(s * 16, 16)] = v
            return carry
        lax.fori_loop(0, nrows // UNROLL, krow, 0)

    for tab_hbm, tail_hbm, out_hbm in ((utabT, utail, urm),
                                       (itabT, itail, irm)):
        def chunk_step(i, carry):
            kc = wid * CPW + i

            @pl.when(kc < NCH)
            def _full():
                off = pl.multiple_of(kc * CW, CW)
                pltpu.sync_copy(tab_hbm.at[:, pl.ds(off, CW)], chunk_v.at[:, pl.ds(0, CW)])
                transpose_rows(CW // PACK)
                pltpu.sync_copy(out_v,
                                out_hbm.at[pl.ds(kc * (CW // PACK), CW // PACK)])

            @pl.when(kc == TAILC)
            def _partial():
                off = pl.multiple_of(TAILC * CW, CW)
                pltpu.sync_copy(tab_hbm.at[:, pl.ds(off, 512)],
                                chunk_v.at[:, pl.ds(0, 512)])
                transpose_rows(512 // PACK)
                pltpu.sync_copy(out_v.at[pl.ds(0, 128)],
                                out_hbm.at[pl.ds(TAILC * (CW // PACK), 128)])
                pltpu.sync_copy(tail_hbm, out_v.at[pl.ds(0, 16)])
                pltpu.sync_copy(out_v.at[pl.ds(0, 16)],
                                out_hbm.at[pl.ds(TROWS - 16, 16)])

            return carry

        lax.fori_loop(0, CPW, chunk_step, 0)


_relayout_call = pl.kernel(
    _relayout_body,
    mesh=plsc.VectorSubcoreMesh(core_axis_name="c", subcore_axis_name="s"),
    compiler_params=_params,
    out_type=[
        jax.ShapeDtypeStruct((TROWS, 128), jnp.float32),
        jax.ShapeDtypeStruct((TROWS, 128), jnp.float32),
    ],
    scratch_types=[
        pltpu.VMEM((DIM, CW + 1), jnp.float32),
        pltpu.VMEM((CW // PACK, 128), jnp.float32),
        pltpu.SemaphoreType.DMA,
    ],
)


def _sc_body(uid_hbm, ids_hbm, utab_hbm, itab_hbm, s_hbm, d0_hbm,
             uidx_v, iidx_v, ubidx_v, ibidx_v, urows_v, irows_v,
             s_v, d0_v, sem):
    wid = lax.axis_index("s") * NC + lax.axis_index("c")
    base = wid * BPW

    pltpu.sync_copy(uid_hbm.at[pl.ds(base, BPW)], uidx_v)
    pltpu.sync_copy(ids_hbm.at[pl.ds(base * NI, BPW * NI)], iidx_v)
    for v in range(BPW // 16):
        ubidx_v[pl.ds(v * 16, 16)] = jnp.right_shift(uidx_v[pl.ds(v * 16, 16)], 2)
    def shift_items(v, carry):
        ibidx_v[pl.ds(v * 16, 16)] = jnp.right_shift(iidx_v[pl.ds(v * 16, 16)], 2)
        return carry
    lax.fori_loop(0, BPW * NI // 16, shift_items, 0)

    iota16 = lax.broadcasted_iota(jnp.int32, (16,), 0)

    for sb in range(NSB):
        copies = [pltpu.async_copy(
            utab_hbm.at[ubidx_v.at[pl.ds(sb * SB, CHUNK)]], urows_v, sem)]
        for c in range(SB * NI // CHUNK):
            copies.append(pltpu.async_copy(
                itab_hbm.at[ibidx_v.at[pl.ds(sb * SB * NI + c * CHUNK, CHUNK)]],
                irows_v.at[pl.ds(c * CHUNK, CHUNK)], sem))
        for cp in copies:
            cp.wait()

        def group(g, carry):
            lrow = g * 16 + iota16
            grow = sb * SB + g * 16 + iota16
            uids = plsc.load_gather(uidx_v, [grow])
            ucol = (uids & 3) * DIM
            icols = []
            irows = []
            for j in range(NI):
                pos = grow * NI + j
                ids_j = plsc.load_gather(iidx_v, [pos])
                icols.append((ids_j & 3) * DIM)
                irows.append(lrow * NI + j)
            acc = [jnp.zeros((16,), jnp.float32) for _ in range(NI)]
            for d in range(DIM):
                u = plsc.load_gather(urows_v, [lrow, ucol + d])
                for j in range(NI):
                    it = plsc.load_gather(irows_v, [irows[j], icols[j] + d])
                    acc[j] = acc[j] + u * it
            ssum = jnp.exp(acc[0])
            for j in range(1, NI):
                ssum = ssum + jnp.exp(acc[j])
            s_v[pl.ds(sb * SB + g * 16, 16)] = ssum
            d0_v[pl.ds(sb * SB + g * 16, 16)] = acc[0]
            return carry

        lax.fori_loop(0, SB // 16, group, 0)

    pltpu.sync_copy(s_v, s_hbm.at[pl.ds(base, BPW)])
    pltpu.sync_copy(d0_v, d0_hbm.at[pl.ds(base, BPW)])


_sc_call = pl.kernel(
    _sc_body,
    mesh=plsc.VectorSubcoreMesh(core_axis_name="c", subcore_axis_name="s"),
    compiler_params=_params,
    out_type=[
        jax.ShapeDtypeStruct((B,), jnp.float32),
        jax.ShapeDtypeStruct((B,), jnp.float32),
    ],
    scratch_types=[
        pltpu.VMEM((BPW,), jnp.int32),
        pltpu.VMEM((BPW * NI,), jnp.int32),
        pltpu.VMEM((BPW,), jnp.int32),
        pltpu.VMEM((BPW * NI,), jnp.int32),
        pltpu.VMEM((SB, 128), jnp.float32),
        pltpu.VMEM((SB * NI, 128), jnp.float32),
        pltpu.VMEM((BPW,), jnp.float32),
        pltpu.VMEM((BPW,), jnp.float32),
        pltpu.SemaphoreType.DMA,
    ],
)


def _tc_loss_body(s_ref, d0_ref, out_ref):
    out_ref[0, 0] = (jnp.sum(jnp.log(s_ref[:])) - jnp.sum(d0_ref[:])) / B


_tc_loss = pl.pallas_call(
    _tc_loss_body,
    out_shape=jax.ShapeDtypeStruct((1, 1), jnp.float32),
    out_specs=pl.BlockSpec(memory_space=pltpu.SMEM),
)


def kernel(userid, itemid, user_feature, item_feature, neg_sample,
           user_table, item_table):
    uid = userid.reshape(B).astype(jnp.int32)
    ids = jnp.concatenate(
        [itemid.astype(jnp.int32), neg_sample.astype(jnp.int32)], axis=1
    ).reshape(B * NI)
    utail = lax.slice(user_table, (NIDS - 64, 0), (NIDS, DIM)).reshape(16, 128)
    itail = lax.slice(item_table, (NIDS - 64, 0), (NIDS, DIM)).reshape(16, 128)
    urm, irm = _relayout_call(user_table.T, item_table.T, utail, itail)
    s, d0 = _sc_call(uid, ids, urm, irm)
    loss = _tc_loss(s.reshape(B // 128, 128), d0.reshape(B // 128, 128))
    return loss[0, 0]


# speedup vs baseline: 2.0379x; 2.0376x over previous
import functools

import jax
import jax.numpy as jnp
from jax import lax
from jax.experimental import pallas as pl
from jax.experimental.pallas import tpu as pltpu
from jax.experimental.pallas import tpu_sc as plsc

B = 16384
DIM = 32
NI = 5
NC = 2
NS = 16
NW = NC * NS
BPW = B // NW
CHUNK = 128
UCH = BPW // CHUNK
ICH = BPW * NI // CHUNK
GROUPS = BPW // 16


def _sc_body(uid_hbm, ids_hbm, utab_hbm, itab_hbm, s_hbm, d0_hbm,
             uidx_v, iidx_v, urows_v, irows_v, s_v, d0_v, sem):
    wid = lax.axis_index("s") * NC + lax.axis_index("c")
    base = wid * BPW

    pltpu.sync_copy(uid_hbm.at[pl.ds(base, BPW)], uidx_v)
    pltpu.sync_copy(ids_hbm.at[pl.ds(base * NI, BPW * NI)], iidx_v)

    copies = []
    for c in range(UCH):
        copies.append(pltpu.async_copy(
            utab_hbm.at[uidx_v.at[pl.ds(c * CHUNK, CHUNK)]],
            urows_v.at[pl.ds(c * CHUNK, CHUNK)], sem))
    for c in range(ICH):
        copies.append(pltpu.async_copy(
            itab_hbm.at[iidx_v.at[pl.ds(c * CHUNK, CHUNK)]],
            irows_v.at[pl.ds(c * CHUNK, CHUNK)], sem))
    for cp in copies:
        cp.wait()

    iota16 = lax.broadcasted_iota(jnp.int32, (16,), 0)
    perms = [jnp.bitwise_xor(iota16, o) for o in (1, 2, 4, 8)]
    masks = [(iota16 & o) == 0 for o in (1, 2, 4, 8)]

    def lane_sums(vregs):
        for st in range(4):
            perm, mask = perms[st], masks[st]
            nxt = []
            for k in range(len(vregs) // 2):
                a, b = vregs[2 * k], vregs[2 * k + 1]
                sa = a + a.at[perm].get(mode="promise_in_bounds")
                sb = b + b.at[perm].get(mode="promise_in_bounds")
                nxt.append(jnp.where(mask, sa, sb))
            vregs = nxt
        return vregs[0]

    def group(g, carry):
        u0 = []
        u1 = []
        for r in range(16):
            row = g * 16 + r
            u0.append(urows_v[row, pl.ds(0, 16)])
            u1.append(urows_v[row, pl.ds(16, 16)])
        dots = []
        for j in range(NI):
            prods = []
            for r in range(16):
                irow = (g * 16 + r) * NI + j
                i0 = irows_v[irow, pl.ds(0, 16)]
                i1 = irows_v[irow, pl.ds(16, 16)]
                prods.append(u0[r] * i0 + u1[r] * i1)
            dots.append(lane_sums(prods))
        ssum = jnp.exp(dots[0])
        for j in range(1, NI):
            ssum = ssum + jnp.exp(dots[j])
        s_v[pl.ds(g * 16, 16)] = ssum
        d0_v[pl.ds(g * 16, 16)] = dots[0]
        return carry

    lax.fori_loop(0, GROUPS, group, 0)

    pltpu.sync_copy(s_v, s_hbm.at[pl.ds(base, BPW)])
    pltpu.sync_copy(d0_v, d0_hbm.at[pl.ds(base, BPW)])


_sc_call = pl.kernel(
    _sc_body,
    mesh=plsc.VectorSubcoreMesh(core_axis_name="c", subcore_axis_name="s"),
    compiler_params=pltpu.CompilerParams(use_tc_tiling_on_sc=False),
    out_type=[
        jax.ShapeDtypeStruct((B,), jnp.float32),
        jax.ShapeDtypeStruct((B,), jnp.float32),
    ],
    scratch_types=[
        pltpu.VMEM((BPW,), jnp.int32),
        pltpu.VMEM((BPW * NI,), jnp.int32),
        pltpu.VMEM((BPW, DIM), jnp.float32),
        pltpu.VMEM((BPW * NI, DIM), jnp.float32),
        pltpu.VMEM((BPW,), jnp.float32),
        pltpu.VMEM((BPW,), jnp.float32),
        pltpu.SemaphoreType.DMA,
    ],
)


def _tc_loss_body(s_ref, d0_ref, out_ref):
    out_ref[0, 0] = (jnp.sum(jnp.log(s_ref[:])) - jnp.sum(d0_ref[:])) / B


_tc_loss = pl.pallas_call(
    _tc_loss_body,
    out_shape=jax.ShapeDtypeStruct((1, 1), jnp.float32),
    out_specs=pl.BlockSpec(memory_space=pltpu.SMEM),
)


def kernel(userid, itemid, user_feature, item_feature, neg_sample,
           user_table, item_table):
    uid = userid.reshape(B).astype(jnp.int32)
    ids = jnp.concatenate(
        [itemid.astype(jnp.int32), neg_sample.astype(jnp.int32)], axis=1
    ).reshape(B * NI)
    s, d0 = _sc_call(uid, ids, user_table, item_table)
    loss = _tc_loss(s.reshape(B // CHUNK, CHUNK), d0.reshape(B // CHUNK, CHUNK))
    return loss[0, 0]
